# Initial kernel scaffold; baseline (speedup 1.0000x reference)
import jax
import jax.numpy as jnp
from jax.experimental import pallas as pl


def kernel(atom_types, frac_coords, lengths, angles, timesteps, emb_table, time_W, time_b, edge_w1, edge_b1, edge_w2, edge_b2, node_w1, node_b1, node_w2, node_b2, coord_w1, coord_b1, coord_w2, coord_b2):
    raise NotImplementedError("write your pallas kernel here")


# speedup vs baseline: 2.4103x; 2.4103x over previous
import functools

import jax
import jax.numpy as jnp
import numpy as np
from jax.experimental import pallas as pl
from jax.experimental.pallas import tpu as pltpu

N_ATOMS = 256
NODE_DIM = 256
EDGE_DIM = 64
HIDDEN = 256
NUM_LAYERS = 4
MAX_NEIGHBORS = 24
CUTOFF = 8.0
NCAND = 27 * N_ATOMS
E = N_ATOMS * MAX_NEIGHBORS
EB = 8
EBS = E // EB
SB = N_ATOMS // EB


def _build_lattice(lengths, angles):
    a, b, c = lengths[0], lengths[1], lengths[2]
    ang = angles * (np.pi / 180.0)
    alpha, beta, gamma = ang[0], ang[1], ang[2]
    lx = a
    xy = b * jnp.cos(gamma)
    xz = c * jnp.cos(beta)
    ly = b * jnp.sin(gamma)
    yz = (b * c * jnp.cos(alpha) - xy * xz) / ly
    lz = jnp.sqrt(c ** 2 - xz ** 2 - yz ** 2)
    z = jnp.zeros_like(lx)
    return jnp.stack([jnp.stack([lx, z, z]), jnp.stack([xy, ly, z]),
                      jnp.stack([xz, yz, lz])])


def _k1_body(sx, sy, sz, cart, oh, emb, ts, tw, tb, dm_out, node0_out):
    cx = cart[:, 0:1]
    cy = cart[:, 1:2]
    cz = cart[:, 2:3]
    dx = sx[...] - cx
    dy = sy[...] - cy
    dz = sz[...] - cz
    d = jnp.sqrt(dx * dx + dy * dy + dz * dz)
    valid = (d < CUTOFF) & (d > 0.01)
    dm_out[...] = jnp.where(valid, d, jnp.inf)
    node0_out[...] = (
        _dote(oh[...], emb[...])
        + _dotd(ts[...], tw[...])
        + tb[...])


def _topk_body(dm, d_out, i_out):
    col = jax.lax.broadcasted_iota(jnp.int32, (N_ATOMS, NCAND), 1)
    tcol = jax.lax.broadcasted_iota(jnp.int32, (N_ATOMS, 128), 1)
    d_out[...] = jnp.zeros((N_ATOMS, 128), jnp.float32)
    i_out[...] = jnp.zeros((N_ATOMS, 128), jnp.int32)

    def step(t, _):
        dmv = dm[...]
        rmin = jnp.min(dmv, axis=1, keepdims=True)
        cand = jnp.where(dmv == rmin, col, jnp.int32(2 * NCAND))
        amin = jnp.min(cand, axis=1, keepdims=True)
        d_out[...] = jnp.where(tcol == t, rmin, d_out[...])
        i_out[...] = jnp.where(tcol == t, amin, i_out[...])
        dm[...] = jnp.where(col == amin, jnp.inf, dmv)
        return 0

    jax.lax.fori_loop(0, MAX_NEIGHBORS, step, 0)


def _silu(x):
    return x * jax.nn.sigmoid(x)


def _dotd(a, b):
    return jnp.dot(a, b, preferred_element_type=jnp.float32,
                   precision=jax.lax.Precision.DEFAULT)


def _dote(a, b):
    return jnp.dot(a, b, preferred_element_type=jnp.float32,
                   precision=jax.lax.Precision.HIGHEST)


def _layers_body(node0, cart, ed, dstc, dstr,
                 ew1, eb1, ew2, eb2, nw1, nb1, nw2, nb2,
                 cw1, cb1, cw2, cb2,
                 node_out, coords_out,
                 node_s, coords_s, aggr_s, cdelta_s):
    l = pl.program_id(0)
    e = pl.program_id(1)

    @pl.when((l == 0) & (e == 0))
    def _():
        node_s[...] = node0[...]
        coords_s[...] = cart[...]

    @pl.when(e == 0)
    def _():
        aggr_s[...] = jnp.zeros_like(aggr_s)
        cdelta_s[...] = jnp.zeros_like(cdelta_s)

    @pl.when(e < EB)
    def _():
        node = node_s[...]
        ns32 = node_s[pl.ds(e * SB, SB), :]
        cs32 = coords_s[pl.ds(e * SB, SB), :]
        coords = coords_s[...]

        srow = jax.lax.broadcasted_iota(jnp.int32, (EBS, SB), 0) // MAX_NEIGHBORS
        scol = jax.lax.broadcasted_iota(jnp.int32, (EBS, SB), 1)
        S = jnp.where(srow == scol, 1.0, 0.0).astype(jnp.float32)
        db = dstc[...].astype(jnp.int32)
        P = jnp.where(
            db == jax.lax.broadcasted_iota(jnp.int32, (EBS, N_ATOMS), 1),
            1.0, 0.0)
        dr = dstr[0].astype(jnp.int32)
        PT = jnp.where(
            dr == jax.lax.broadcasted_iota(jnp.int32, (N_ATOMS, EBS), 0),
            1.0, 0.0)

        nsrc = _dote(S, ns32)
        ndst = _dote(P, node)

        dd = ed[...]
        step = np.float32(CUTOFF) / np.float32(EDGE_DIM - 1)
        centers = jax.lax.broadcasted_iota(
            jnp.int32, (1, EDGE_DIM), 1).astype(jnp.float32) * step
        w2 = 2.0 * (CUTOFF / EDGE_DIM) ** 2
        rbf = jnp.exp(-((dd - centers) ** 2) / w2)
        env = 0.5 * (jnp.cos(dd * np.pi / CUTOFF) + 1.0) * (
            dd < CUTOFF).astype(jnp.float32)
        ea = rbf * env

        w1 = ew1[0]
        m = (_dotd(nsrc, w1[:NODE_DIM])
             + _dotd(ndst, w1[NODE_DIM:2 * NODE_DIM])
             + _dotd(ea, w1[2 * NODE_DIM:])
             + eb1[0])
        m = _silu(m)
        m = _silu(_dotd(m, ew2[0]) + eb2[0])

        aggr_s[...] += _dote(PT, m)

        cw = _silu(_dotd(m, cw1[0]) + cb1[0])
        cw = _dotd(cw, cw2[0]) + cb2[0]

        csrc = _dote(S, cs32)
        cdst = _dote(P, coords)
        cd = csrc - cdst
        nrm = jnp.sqrt(jnp.sum(cd * cd, axis=1, keepdims=True))
        cdn = cd / (nrm + 1e-08)
        cdelta_s[...] += _dote(PT, cw * cdn)

    @pl.when(e == EB)
    def _():
        node = node_s[...]
        aggr = aggr_s[...]
        h = (_dotd(node, nw1[0, :NODE_DIM])
             + _dotd(aggr, nw1[0, NODE_DIM:])
             + nb1[0])
        nu = _dotd(_silu(h), nw2[0]) + nb2[0]
        node_s[...] = node + nu
        coords_s[...] = coords_s[...] + cdelta_s[...]

        @pl.when(l == NUM_LAYERS - 1)
        def _():
            node_out[...] = node_s[...]
            coords_out[...] = coords_s[...]


def kernel(atom_types, frac_coords, lengths, angles, timesteps, emb_table,
           time_W, time_b, edge_w1, edge_b1, edge_w2, edge_b2, node_w1,
           node_b1, node_w2, node_b2, coord_w1, coord_b1, coord_w2, coord_b2):
    f32 = jnp.float32
    lattice = _build_lattice(lengths, angles)
    cart = frac_coords @ lattice
    shifts = jnp.asarray(
        [[i, j, k] for i in (-1, 0, 1) for j in (-1, 0, 1) for k in (-1, 0, 1)],
        dtype=f32)
    shiftL = shifts @ lattice
    shifted = (cart[None, :, :] + shiftL[:, None, :]).reshape(NCAND, 3)
    sx = shifted[:, 0].reshape(1, NCAND)
    sy = shifted[:, 1].reshape(1, NCAND)
    sz = shifted[:, 2].reshape(1, NCAND)

    oh = (atom_types[:, None] == jnp.arange(128)[None, :]).astype(f32)
    emb128 = jnp.zeros((128, NODE_DIM), f32).at[:100].set(emb_table)

    dm, node0 = pl.pallas_call(
        _k1_body,
        out_shape=(jax.ShapeDtypeStruct((N_ATOMS, NCAND), f32),
                   jax.ShapeDtypeStruct((N_ATOMS, NODE_DIM), f32)),
    )(sx, sy, sz, cart, oh, emb128, timesteps, time_W, time_b.reshape(1, -1))

    dsel, isel = pl.pallas_call(
        _topk_body,
        out_shape=(jax.ShapeDtypeStruct((N_ATOMS, 128), f32),
                   jax.ShapeDtypeStruct((N_ATOMS, 128), jnp.int32)),
    )(dm)

    edist = dsel[:, :MAX_NEIGHBORS].reshape(E, 1)
    dst = (isel[:, :MAX_NEIGHBORS].reshape(-1) % N_ATOMS).astype(f32)
    dstc = dst.reshape(E, 1)
    dstr = dst.reshape(EB, 1, EBS)

    grid = (NUM_LAYERS, EB + 1)
    eb_map = lambda l, e: (jnp.minimum(e, EB - 1), 0)
    wmap2 = lambda l, e: (l, 0)
    wmap3 = lambda l, e: (l, 0, 0)
    cmap2 = lambda l, e: (0, 0)

    node_out, coords_out = pl.pallas_call(
        _layers_body,
        grid=grid,
        in_specs=[
            pl.BlockSpec((N_ATOMS, NODE_DIM), cmap2),
            pl.BlockSpec((N_ATOMS, 3), cmap2),
            pl.BlockSpec((EBS, 1), eb_map),
            pl.BlockSpec((EBS, 1), eb_map),
            pl.BlockSpec((1, 1, EBS), lambda l, e: (jnp.minimum(e, EB - 1), 0, 0)),
            pl.BlockSpec((1, 2 * NODE_DIM + EDGE_DIM, HIDDEN), wmap3),
            pl.BlockSpec((1, 1, HIDDEN), wmap3),
            pl.BlockSpec((1, HIDDEN, HIDDEN), wmap3),
            pl.BlockSpec((1, 1, HIDDEN), wmap3),
            pl.BlockSpec((1, NODE_DIM + HIDDEN, HIDDEN), wmap3),
            pl.BlockSpec((1, 1, HIDDEN), wmap3),
            pl.BlockSpec((1, HIDDEN, NODE_DIM), wmap3),
            pl.BlockSpec((1, 1, NODE_DIM), wmap3),
            pl.BlockSpec((1, HIDDEN, HIDDEN // 2), wmap3),
            pl.BlockSpec((1, 1, HIDDEN // 2), wmap3),
            pl.BlockSpec((1, HIDDEN // 2, 1), wmap3),
            pl.BlockSpec((1, 1, 1), wmap3),
        ],
        out_specs=[
            pl.BlockSpec((N_ATOMS, NODE_DIM), cmap2),
            pl.BlockSpec((N_ATOMS, 3), cmap2),
        ],
        out_shape=(jax.ShapeDtypeStruct((N_ATOMS, NODE_DIM), f32),
                   jax.ShapeDtypeStruct((N_ATOMS, 3), f32)),
        scratch_shapes=[
            pltpu.VMEM((N_ATOMS, NODE_DIM), f32),
            pltpu.VMEM((N_ATOMS, 3), f32),
            pltpu.VMEM((N_ATOMS, NODE_DIM), f32),
            pltpu.VMEM((N_ATOMS, 3), f32),
        ],
        compiler_params=pltpu.CompilerParams(
            dimension_semantics=("arbitrary", "arbitrary")),
    )(node0, cart, edist, dstc, dstr,
      edge_w1, edge_b1.reshape(NUM_LAYERS, 1, HIDDEN),
      edge_w2, edge_b2.reshape(NUM_LAYERS, 1, HIDDEN),
      node_w1, node_b1.reshape(NUM_LAYERS, 1, HIDDEN),
      node_w2, node_b2.reshape(NUM_LAYERS, 1, NODE_DIM),
      coord_w1, coord_b1.reshape(NUM_LAYERS, 1, HIDDEN // 2),
      coord_w2, coord_b2.reshape(NUM_LAYERS, 1, 1))

    return node_out, coords_out


# speedup vs baseline: 3.2837x; 1.3624x over previous
import functools

import jax
import jax.numpy as jnp
import numpy as np
from jax.experimental import pallas as pl
from jax.experimental.pallas import tpu as pltpu

N_ATOMS = 256
NODE_DIM = 256
EDGE_DIM = 64
HIDDEN = 256
NUM_LAYERS = 4
MAX_NEIGHBORS = 24
CUTOFF = 8.0
NCAND = 27 * N_ATOMS
E = N_ATOMS * MAX_NEIGHBORS
EB = 8
EBS = E // EB
SB = N_ATOMS // EB


def _build_lattice(lengths, angles):
    a, b, c = lengths[0], lengths[1], lengths[2]
    ang = angles * (np.pi / 180.0)
    alpha, beta, gamma = ang[0], ang[1], ang[2]
    lx = a
    xy = b * jnp.cos(gamma)
    xz = c * jnp.cos(beta)
    ly = b * jnp.sin(gamma)
    yz = (b * c * jnp.cos(alpha) - xy * xz) / ly
    lz = jnp.sqrt(c ** 2 - xz ** 2 - yz ** 2)
    z = jnp.zeros_like(lx)
    return jnp.stack([jnp.stack([lx, z, z]), jnp.stack([xy, ly, z]),
                      jnp.stack([xz, yz, lz])])


def _k1_body(sx, sy, sz, cart, oh, emb, ts, tw, tb, dm_out, node0_out):
    cx = cart[:, 0:1]
    cy = cart[:, 1:2]
    cz = cart[:, 2:3]
    dx = sx[...] - cx
    dy = sy[...] - cy
    dz = sz[...] - cz
    d = jnp.sqrt(dx * dx + dy * dy + dz * dz)
    valid = (d < CUTOFF) & (d > 0.01)
    dm_out[...] = jnp.where(valid, d, jnp.inf)
    node0_out[...] = (
        _dote(oh[...], emb[...])
        + _dotd(ts[...], tw[...])
        + tb[...])


def _topk_body(dm, d_out, i_out):
    col = jax.lax.broadcasted_iota(jnp.int32, (N_ATOMS, NCAND), 1)
    tcol = jax.lax.broadcasted_iota(jnp.int32, (N_ATOMS, 128), 1)
    d_out[...] = jnp.zeros((N_ATOMS, 128), jnp.float32)
    i_out[...] = jnp.zeros((N_ATOMS, 128), jnp.int32)

    def step(t, _):
        dmv = dm[...]
        rmin = jnp.min(dmv, axis=1, keepdims=True)
        cand = jnp.where(dmv == rmin, col, jnp.int32(2 * NCAND))
        amin = jnp.min(cand, axis=1, keepdims=True)
        d_out[...] = jnp.where(tcol == t, rmin, d_out[...])
        i_out[...] = jnp.where(tcol == t, amin, i_out[...])
        dm[...] = jnp.where(col == amin, jnp.inf, dmv)
        return 0

    jax.lax.fori_loop(0, MAX_NEIGHBORS, step, 0)


def _silu(x):
    return x * jax.nn.sigmoid(x)


def _dotd(a, b):
    return jnp.dot(a, b, preferred_element_type=jnp.float32,
                   precision=jax.lax.Precision.DEFAULT)


def _dote(sel, v):
    m16 = jnp.int32(-65536)
    vi = jax.lax.bitcast_convert_type(v, jnp.int32)
    h1 = jax.lax.bitcast_convert_type(vi & m16, jnp.float32)
    r1 = v - h1
    r1i = jax.lax.bitcast_convert_type(r1, jnp.int32)
    h2 = jax.lax.bitcast_convert_type(r1i & m16, jnp.float32)
    r2 = r1 - h2
    return _dotd(sel, h1) + _dotd(sel, h2) + _dotd(sel, r2)


def _layers_body(node0, cart, ed, dstc, dstr,
                 ew1, eb1, ew2, eb2, nw1, nb1, nw2, nb2,
                 cw1, cb1, cw2, cb2,
                 node_out, coords_out,
                 node_s, coords_s, aggr_s, cdelta_s):
    l = pl.program_id(0)
    e = pl.program_id(1)

    @pl.when((l == 0) & (e == 0))
    def _():
        node_s[...] = node0[...]
        coords_s[...] = cart[...]

    @pl.when(e == 0)
    def _():
        aggr_s[...] = jnp.zeros_like(aggr_s)
        cdelta_s[...] = jnp.zeros_like(cdelta_s)

    @pl.when(e < EB)
    def _():
        node = node_s[...]
        ns32 = node_s[pl.ds(e * SB, SB), :]
        cs32 = coords_s[pl.ds(e * SB, SB), :]
        coords = coords_s[...]

        srow = jax.lax.broadcasted_iota(jnp.int32, (EBS, SB), 0) // MAX_NEIGHBORS
        scol = jax.lax.broadcasted_iota(jnp.int32, (EBS, SB), 1)
        S = jnp.where(srow == scol, 1.0, 0.0).astype(jnp.float32)
        db = dstc[...].astype(jnp.int32)
        P = jnp.where(
            db == jax.lax.broadcasted_iota(jnp.int32, (EBS, N_ATOMS), 1),
            1.0, 0.0)
        dr = dstr[0].astype(jnp.int32)
        PT = jnp.where(
            dr == jax.lax.broadcasted_iota(jnp.int32, (N_ATOMS, EBS), 0),
            1.0, 0.0)

        nsrc = _dote(S, ns32)
        ndst = _dote(P, node)

        dd = ed[...]
        step = np.float32(CUTOFF) / np.float32(EDGE_DIM - 1)
        centers = jax.lax.broadcasted_iota(
            jnp.int32, (1, EDGE_DIM), 1).astype(jnp.float32) * step
        w2 = 2.0 * (CUTOFF / EDGE_DIM) ** 2
        rbf = jnp.exp(-((dd - centers) ** 2) / w2)
        env = 0.5 * (jnp.cos(dd * np.pi / CUTOFF) + 1.0) * (
            dd < CUTOFF).astype(jnp.float32)
        ea = rbf * env

        w1 = ew1[0]
        m = (_dotd(nsrc, w1[:NODE_DIM])
             + _dotd(ndst, w1[NODE_DIM:2 * NODE_DIM])
             + _dotd(ea, w1[2 * NODE_DIM:])
             + eb1[0])
        m = _silu(m)
        m = _silu(_dotd(m, ew2[0]) + eb2[0])

        aggr_s[...] += _dote(PT, m)

        cw = _silu(_dotd(m, cw1[0]) + cb1[0])
        cw = _dotd(cw, cw2[0]) + cb2[0]

        csrc = _dote(S, cs32)
        cdst = _dote(P, coords)
        cd = csrc - cdst
        nrm = jnp.sqrt(jnp.sum(cd * cd, axis=1, keepdims=True))
        cdn = cd / (nrm + 1e-08)
        cdelta_s[...] += _dote(PT, cw * cdn)

    @pl.when(e == EB)
    def _():
        node = node_s[...]
        aggr = aggr_s[...]
        h = (_dotd(node, nw1[0, :NODE_DIM])
             + _dotd(aggr, nw1[0, NODE_DIM:])
             + nb1[0])
        nu = _dotd(_silu(h), nw2[0]) + nb2[0]
        node_s[...] = node + nu
        coords_s[...] = coords_s[...] + cdelta_s[...]

        @pl.when(l == NUM_LAYERS - 1)
        def _():
            node_out[...] = node_s[...]
            coords_out[...] = coords_s[...]


def kernel(atom_types, frac_coords, lengths, angles, timesteps, emb_table,
           time_W, time_b, edge_w1, edge_b1, edge_w2, edge_b2, node_w1,
           node_b1, node_w2, node_b2, coord_w1, coord_b1, coord_w2, coord_b2):
    f32 = jnp.float32
    lattice = _build_lattice(lengths, angles)
    cart = frac_coords @ lattice
    shifts = jnp.asarray(
        [[i, j, k] for i in (-1, 0, 1) for j in (-1, 0, 1) for k in (-1, 0, 1)],
        dtype=f32)
    shiftL = shifts @ lattice
    shifted = (cart[None, :, :] + shiftL[:, None, :]).reshape(NCAND, 3)
    sx = shifted[:, 0].reshape(1, NCAND)
    sy = shifted[:, 1].reshape(1, NCAND)
    sz = shifted[:, 2].reshape(1, NCAND)

    oh = (atom_types[:, None] == jnp.arange(128)[None, :]).astype(f32)
    emb128 = jnp.zeros((128, NODE_DIM), f32).at[:100].set(emb_table)

    dm, node0 = pl.pallas_call(
        _k1_body,
        out_shape=(jax.ShapeDtypeStruct((N_ATOMS, NCAND), f32),
                   jax.ShapeDtypeStruct((N_ATOMS, NODE_DIM), f32)),
    )(sx, sy, sz, cart, oh, emb128, timesteps, time_W, time_b.reshape(1, -1))

    dsel, isel = pl.pallas_call(
        _topk_body,
        out_shape=(jax.ShapeDtypeStruct((N_ATOMS, 128), f32),
                   jax.ShapeDtypeStruct((N_ATOMS, 128), jnp.int32)),
    )(dm)

    edist = dsel[:, :MAX_NEIGHBORS].reshape(E, 1)
    dst = (isel[:, :MAX_NEIGHBORS].reshape(-1) % N_ATOMS).astype(f32)
    dstc = dst.reshape(E, 1)
    dstr = dst.reshape(EB, 1, EBS)

    grid = (NUM_LAYERS, EB + 1)
    eb_map = lambda l, e: (jnp.minimum(e, EB - 1), 0)
    wmap2 = lambda l, e: (l, 0)
    wmap3 = lambda l, e: (l, 0, 0)
    cmap2 = lambda l, e: (0, 0)

    node_out, coords_out = pl.pallas_call(
        _layers_body,
        grid=grid,
        in_specs=[
            pl.BlockSpec((N_ATOMS, NODE_DIM), cmap2),
            pl.BlockSpec((N_ATOMS, 3), cmap2),
            pl.BlockSpec((EBS, 1), eb_map),
            pl.BlockSpec((EBS, 1), eb_map),
            pl.BlockSpec((1, 1, EBS), lambda l, e: (jnp.minimum(e, EB - 1), 0, 0)),
            pl.BlockSpec((1, 2 * NODE_DIM + EDGE_DIM, HIDDEN), wmap3),
            pl.BlockSpec((1, 1, HIDDEN), wmap3),
            pl.BlockSpec((1, HIDDEN, HIDDEN), wmap3),
            pl.BlockSpec((1, 1, HIDDEN), wmap3),
            pl.BlockSpec((1, NODE_DIM + HIDDEN, HIDDEN), wmap3),
            pl.BlockSpec((1, 1, HIDDEN), wmap3),
            pl.BlockSpec((1, HIDDEN, NODE_DIM), wmap3),
            pl.BlockSpec((1, 1, NODE_DIM), wmap3),
            pl.BlockSpec((1, HIDDEN, HIDDEN // 2), wmap3),
            pl.BlockSpec((1, 1, HIDDEN // 2), wmap3),
            pl.BlockSpec((1, HIDDEN // 2, 1), wmap3),
            pl.BlockSpec((1, 1, 1), wmap3),
        ],
        out_specs=[
            pl.BlockSpec((N_ATOMS, NODE_DIM), cmap2),
            pl.BlockSpec((N_ATOMS, 3), cmap2),
        ],
        out_shape=(jax.ShapeDtypeStruct((N_ATOMS, NODE_DIM), f32),
                   jax.ShapeDtypeStruct((N_ATOMS, 3), f32)),
        scratch_shapes=[
            pltpu.VMEM((N_ATOMS, NODE_DIM), f32),
            pltpu.VMEM((N_ATOMS, 3), f32),
            pltpu.VMEM((N_ATOMS, NODE_DIM), f32),
            pltpu.VMEM((N_ATOMS, 3), f32),
        ],
        compiler_params=pltpu.CompilerParams(
            dimension_semantics=("arbitrary", "arbitrary")),
    )(node0, cart, edist, dstc, dstr,
      edge_w1, edge_b1.reshape(NUM_LAYERS, 1, HIDDEN),
      edge_w2, edge_b2.reshape(NUM_LAYERS, 1, HIDDEN),
      node_w1, node_b1.reshape(NUM_LAYERS, 1, HIDDEN),
      node_w2, node_b2.reshape(NUM_LAYERS, 1, NODE_DIM),
      coord_w1, coord_b1.reshape(NUM_LAYERS, 1, HIDDEN // 2),
      coord_w2, coord_b2.reshape(NUM_LAYERS, 1, 1))

    return node_out, coords_out


# speedup vs baseline: 3.5611x; 1.0845x over previous
import functools

import jax
import jax.numpy as jnp
import numpy as np
from jax.experimental import pallas as pl
from jax.experimental.pallas import tpu as pltpu

N_ATOMS = 256
NODE_DIM = 256
EDGE_DIM = 64
HIDDEN = 256
NUM_LAYERS = 4
MAX_NEIGHBORS = 24
CUTOFF = 8.0
NCAND = 27 * N_ATOMS
E = N_ATOMS * MAX_NEIGHBORS
EB = 8
EBS = E // EB
SB = N_ATOMS // EB


def _build_lattice(lengths, angles):
    a, b, c = lengths[0], lengths[1], lengths[2]
    ang = angles * (np.pi / 180.0)
    alpha, beta, gamma = ang[0], ang[1], ang[2]
    lx = a
    xy = b * jnp.cos(gamma)
    xz = c * jnp.cos(beta)
    ly = b * jnp.sin(gamma)
    yz = (b * c * jnp.cos(alpha) - xy * xz) / ly
    lz = jnp.sqrt(c ** 2 - xz ** 2 - yz ** 2)
    z = jnp.zeros_like(lx)
    return jnp.stack([jnp.stack([lx, z, z]), jnp.stack([xy, ly, z]),
                      jnp.stack([xz, yz, lz])])


def _k1_body(sx, sy, sz, cart, oh, emb, ts, tw, tb, dm_out, node0_out):
    cx = cart[:, 0:1]
    cy = cart[:, 1:2]
    cz = cart[:, 2:3]
    dx = sx[...] - cx
    dy = sy[...] - cy
    dz = sz[...] - cz
    d = jnp.sqrt(dx * dx + dy * dy + dz * dz)
    valid = (d < CUTOFF) & (d > 0.01)
    dm_out[...] = jnp.where(valid, d, jnp.inf)
    node0_out[...] = (
        _dote(oh[...], emb[...])
        + _dotd(ts[...], tw[...])
        + tb[...])


def _topk_body(dm, d_out, i_out):
    col = jax.lax.broadcasted_iota(jnp.int32, (N_ATOMS, NCAND), 1)
    tcol = jax.lax.broadcasted_iota(jnp.int32, (N_ATOMS, 128), 1)
    d_out[...] = jnp.zeros((N_ATOMS, 128), jnp.float32)
    i_out[...] = jnp.zeros((N_ATOMS, 128), jnp.int32)

    def step(t, _):
        dmv = dm[...]
        rmin = jnp.min(dmv, axis=1, keepdims=True)
        cand = jnp.where(dmv == rmin, col, jnp.int32(2 * NCAND))
        amin = jnp.min(cand, axis=1, keepdims=True)
        d_out[...] = jnp.where(tcol == t, rmin, d_out[...])
        i_out[...] = jnp.where(tcol == t, amin, i_out[...])
        dm[...] = jnp.where(col == amin, jnp.inf, dmv)
        return 0

    jax.lax.fori_loop(0, MAX_NEIGHBORS, step, 0)


def _silu(x):
    return x * jax.nn.sigmoid(x)


def _dotd(a, b):
    return jnp.dot(a, b, preferred_element_type=jnp.float32,
                   precision=jax.lax.Precision.DEFAULT)


def _dote(sel, v):
    m16 = jnp.int32(-65536)
    vi = jax.lax.bitcast_convert_type(v, jnp.int32)
    h1 = jax.lax.bitcast_convert_type(vi & m16, jnp.float32)
    r1 = v - h1
    r1i = jax.lax.bitcast_convert_type(r1, jnp.int32)
    h2 = jax.lax.bitcast_convert_type(r1i & m16, jnp.float32)
    r2 = r1 - h2
    return _dotd(sel, h1) + _dotd(sel, h2) + _dotd(sel, r2)


def _layers_body(node0, cart, ed, dstc, dstr,
                 ew1, eb1, ew2, eb2, nw1, nb1, nw2, nb2,
                 cw1, cb1, cw2, cb2,
                 node_out, coords_out,
                 node_s, coords_s, aggr_s, cdelta_s, ea_s):
    l = pl.program_id(0)
    e = pl.program_id(1)

    @pl.when((l == 0) & (e == 0))
    def _():
        node_s[...] = node0[...]
        coords_s[...] = cart[...]

    @pl.when(e == 0)
    def _():
        aggr_s[...] = jnp.zeros_like(aggr_s)
        cdelta_s[...] = jnp.zeros_like(cdelta_s)

    @pl.when(e < EB)
    def _():
        node = node_s[...]
        ns32 = node_s[pl.ds(e * SB, SB), :]
        cs32 = coords_s[pl.ds(e * SB, SB), :]
        coords = coords_s[...]

        srow = jax.lax.broadcasted_iota(jnp.int32, (EBS, SB), 0) // MAX_NEIGHBORS
        scol = jax.lax.broadcasted_iota(jnp.int32, (EBS, SB), 1)
        S = jnp.where(srow == scol, 1.0, 0.0).astype(jnp.float32)
        db = dstc[...].astype(jnp.int32)
        P = jnp.where(
            db == jax.lax.broadcasted_iota(jnp.int32, (EBS, N_ATOMS), 1),
            1.0, 0.0)
        dr = dstr[0].astype(jnp.int32)
        PT = jnp.where(
            dr == jax.lax.broadcasted_iota(jnp.int32, (N_ATOMS, EBS), 0),
            1.0, 0.0)

        nsrc = _dote(S, ns32)
        ndst = _dote(P, node)

        @pl.when(l == 0)
        def _():
            dd = ed[...]
            step = np.float32(CUTOFF) / np.float32(EDGE_DIM - 1)
            centers = jax.lax.broadcasted_iota(
                jnp.int32, (EBS, EDGE_DIM), 1).astype(jnp.float32) * step
            w2 = 2.0 * (CUTOFF / EDGE_DIM) ** 2
            rbf = jnp.exp(-((dd - centers) ** 2) / w2)
            env = 0.5 * (jnp.cos(dd * np.pi / CUTOFF) + 1.0) * (
                dd < CUTOFF).astype(jnp.float32)
            ea_s[pl.ds(e * EBS, EBS), :] = rbf * env

        ea = ea_s[pl.ds(e * EBS, EBS), :]

        w1 = ew1[0]
        m = (_dotd(nsrc, w1[:NODE_DIM])
             + _dotd(ndst, w1[NODE_DIM:2 * NODE_DIM])
             + _dotd(ea, w1[2 * NODE_DIM:])
             + eb1[0])
        m = _silu(m)
        m = _silu(_dotd(m, ew2[0]) + eb2[0])

        aggr_s[...] += _dote(PT, m)

        cw = _silu(_dotd(m, cw1[0]) + cb1[0])
        cw = _dotd(cw, cw2[0]) + cb2[0]

        csrc = _dote(S, cs32)
        cdst = _dote(P, coords)
        cd = csrc - cdst
        nrm = jnp.sqrt(jnp.sum(cd * cd, axis=1, keepdims=True))
        cdn = cd / (nrm + 1e-08)
        cdelta_s[...] += _dote(PT, cw * cdn)

    @pl.when(e == EB)
    def _():
        node = node_s[...]
        aggr = aggr_s[...]
        h = (_dotd(node, nw1[0, :NODE_DIM])
             + _dotd(aggr, nw1[0, NODE_DIM:])
             + nb1[0])
        nu = _dotd(_silu(h), nw2[0]) + nb2[0]
        node_s[...] = node + nu
        coords_s[...] = coords_s[...] + cdelta_s[...]

        @pl.when(l == NUM_LAYERS - 1)
        def _():
            node_out[...] = node_s[...]
            coords_out[...] = coords_s[...]


def kernel(atom_types, frac_coords, lengths, angles, timesteps, emb_table,
           time_W, time_b, edge_w1, edge_b1, edge_w2, edge_b2, node_w1,
           node_b1, node_w2, node_b2, coord_w1, coord_b1, coord_w2, coord_b2):
    f32 = jnp.float32
    lattice = _build_lattice(lengths, angles)
    cart = frac_coords @ lattice
    shifts = jnp.asarray(
        [[i, j, k] for i in (-1, 0, 1) for j in (-1, 0, 1) for k in (-1, 0, 1)],
        dtype=f32)
    shiftL = shifts @ lattice
    shifted = (cart[None, :, :] + shiftL[:, None, :]).reshape(NCAND, 3)
    sx = shifted[:, 0].reshape(1, NCAND)
    sy = shifted[:, 1].reshape(1, NCAND)
    sz = shifted[:, 2].reshape(1, NCAND)

    oh = (atom_types[:, None] == jnp.arange(128)[None, :]).astype(f32)
    emb128 = jnp.zeros((128, NODE_DIM), f32).at[:100].set(emb_table)

    dm, node0 = pl.pallas_call(
        _k1_body,
        out_shape=(jax.ShapeDtypeStruct((N_ATOMS, NCAND), f32),
                   jax.ShapeDtypeStruct((N_ATOMS, NODE_DIM), f32)),
    )(sx, sy, sz, cart, oh, emb128, timesteps, time_W, time_b.reshape(1, -1))

    dsel, isel = pl.pallas_call(
        _topk_body,
        out_shape=(jax.ShapeDtypeStruct((N_ATOMS, 128), f32),
                   jax.ShapeDtypeStruct((N_ATOMS, 128), jnp.int32)),
    )(dm)

    edist = jnp.broadcast_to(dsel[:, :MAX_NEIGHBORS].reshape(E, 1),
                             (E, EDGE_DIM))
    dst = (isel[:, :MAX_NEIGHBORS].reshape(-1) % N_ATOMS).astype(f32)
    dstc = dst.reshape(E, 1)
    dstr = dst.reshape(EB, 1, EBS)

    grid = (NUM_LAYERS, EB + 1)
    eb_map = lambda l, e: (jnp.minimum(e, EB - 1), 0)
    wmap2 = lambda l, e: (l, 0)
    wmap3 = lambda l, e: (l, 0, 0)
    cmap2 = lambda l, e: (0, 0)

    node_out, coords_out = pl.pallas_call(
        _layers_body,
        grid=grid,
        in_specs=[
            pl.BlockSpec((N_ATOMS, NODE_DIM), cmap2),
            pl.BlockSpec((N_ATOMS, 3), cmap2),
            pl.BlockSpec((EBS, EDGE_DIM), eb_map),
            pl.BlockSpec((EBS, 1), eb_map),
            pl.BlockSpec((1, 1, EBS), lambda l, e: (jnp.minimum(e, EB - 1), 0, 0)),
            pl.BlockSpec((1, 2 * NODE_DIM + EDGE_DIM, HIDDEN), wmap3),
            pl.BlockSpec((1, 1, HIDDEN), wmap3),
            pl.BlockSpec((1, HIDDEN, HIDDEN), wmap3),
            pl.BlockSpec((1, 1, HIDDEN), wmap3),
            pl.BlockSpec((1, NODE_DIM + HIDDEN, HIDDEN), wmap3),
            pl.BlockSpec((1, 1, HIDDEN), wmap3),
            pl.BlockSpec((1, HIDDEN, NODE_DIM), wmap3),
            pl.BlockSpec((1, 1, NODE_DIM), wmap3),
            pl.BlockSpec((1, HIDDEN, HIDDEN // 2), wmap3),
            pl.BlockSpec((1, 1, HIDDEN // 2), wmap3),
            pl.BlockSpec((1, HIDDEN // 2, 1), wmap3),
            pl.BlockSpec((1, 1, 1), wmap3),
        ],
        out_specs=[
            pl.BlockSpec((N_ATOMS, NODE_DIM), cmap2),
            pl.BlockSpec((N_ATOMS, 3), cmap2),
        ],
        out_shape=(jax.ShapeDtypeStruct((N_ATOMS, NODE_DIM), f32),
                   jax.ShapeDtypeStruct((N_ATOMS, 3), f32)),
        scratch_shapes=[
            pltpu.VMEM((N_ATOMS, NODE_DIM), f32),
            pltpu.VMEM((N_ATOMS, 3), f32),
            pltpu.VMEM((N_ATOMS, NODE_DIM), f32),
            pltpu.VMEM((N_ATOMS, 3), f32),
            pltpu.VMEM((E, EDGE_DIM), f32),
        ],
        compiler_params=pltpu.CompilerParams(
            dimension_semantics=("arbitrary", "arbitrary")),
    )(node0, cart, edist, dstc, dstr,
      edge_w1, edge_b1.reshape(NUM_LAYERS, 1, HIDDEN),
      edge_w2, edge_b2.reshape(NUM_LAYERS, 1, HIDDEN),
      node_w1, node_b1.reshape(NUM_LAYERS, 1, HIDDEN),
      node_w2, node_b2.reshape(NUM_LAYERS, 1, NODE_DIM),
      coord_w1, coord_b1.reshape(NUM_LAYERS, 1, HIDDEN // 2),
      coord_w2, coord_b2.reshape(NUM_LAYERS, 1, 1))

    return node_out, coords_out


# speedup vs baseline: 3.5986x; 1.0105x over previous
import functools

import jax
import jax.numpy as jnp
import numpy as np
from jax.experimental import pallas as pl
from jax.experimental.pallas import tpu as pltpu

N_ATOMS = 256
NODE_DIM = 256
EDGE_DIM = 64
HIDDEN = 256
NUM_LAYERS = 4
MAX_NEIGHBORS = 24
CUTOFF = 8.0
NCAND = 27 * N_ATOMS
E = N_ATOMS * MAX_NEIGHBORS
EB = 8
EBS = E // EB
SB = N_ATOMS // EB


def _build_lattice(lengths, angles):
    a, b, c = lengths[0], lengths[1], lengths[2]
    ang = angles * (np.pi / 180.0)
    alpha, beta, gamma = ang[0], ang[1], ang[2]
    lx = a
    xy = b * jnp.cos(gamma)
    xz = c * jnp.cos(beta)
    ly = b * jnp.sin(gamma)
    yz = (b * c * jnp.cos(alpha) - xy * xz) / ly
    lz = jnp.sqrt(c ** 2 - xz ** 2 - yz ** 2)
    z = jnp.zeros_like(lx)
    return jnp.stack([jnp.stack([lx, z, z]), jnp.stack([xy, ly, z]),
                      jnp.stack([xz, yz, lz])])


def _k1_body(sx, sy, sz, cart, oh, emb, ts, tw, tb, dm_out, node0_out):
    cx = cart[:, 0:1]
    cy = cart[:, 1:2]
    cz = cart[:, 2:3]
    dx = sx[...] - cx
    dy = sy[...] - cy
    dz = sz[...] - cz
    d = jnp.sqrt(dx * dx + dy * dy + dz * dz)
    valid = (d < CUTOFF) & (d > 0.01)
    dm_out[...] = jnp.where(valid, d, jnp.inf)
    node0_out[...] = (
        _dote(oh[...], emb[...])
        + _dotd(ts[...], tw[...])
        + tb[...])


def _topk_body(dm, d_out, i_out):
    col = jax.lax.broadcasted_iota(jnp.int32, (N_ATOMS, NCAND), 1)
    tcol = jax.lax.broadcasted_iota(jnp.int32, (N_ATOMS, 128), 1)
    d_out[...] = jnp.zeros((N_ATOMS, 128), jnp.float32)
    i_out[...] = jnp.zeros((N_ATOMS, 128), jnp.int32)

    def step(t, _):
        dmv = dm[...]
        r1 = jnp.min(dmv, axis=1, keepdims=True)
        c1 = jnp.where(dmv == r1, col, jnp.int32(2 * NCAND))
        a1 = jnp.min(c1, axis=1, keepdims=True)
        dmv2 = jnp.where(col == a1, jnp.inf, dmv)
        r2 = jnp.min(dmv2, axis=1, keepdims=True)
        c2 = jnp.where(dmv2 == r2, col, jnp.int32(2 * NCAND))
        a2 = jnp.min(c2, axis=1, keepdims=True)
        d_out[...] = jnp.where(tcol == 2 * t, r1, d_out[...])
        i_out[...] = jnp.where(tcol == 2 * t, a1, i_out[...])
        d_out[...] = jnp.where(tcol == 2 * t + 1, r2, d_out[...])
        i_out[...] = jnp.where(tcol == 2 * t + 1, a2, i_out[...])
        dm[...] = jnp.where(col == a2, jnp.inf, dmv2)
        return 0

    jax.lax.fori_loop(0, MAX_NEIGHBORS // 2, step, 0)


def _silu(x):
    return x * jax.nn.sigmoid(x)


def _dotd(a, b):
    return jnp.dot(a, b, preferred_element_type=jnp.float32,
                   precision=jax.lax.Precision.DEFAULT)


def _dote(sel, v):
    m16 = jnp.int32(-65536)
    vi = jax.lax.bitcast_convert_type(v, jnp.int32)
    h1 = jax.lax.bitcast_convert_type(vi & m16, jnp.float32)
    r1 = v - h1
    r1i = jax.lax.bitcast_convert_type(r1, jnp.int32)
    h2 = jax.lax.bitcast_convert_type(r1i & m16, jnp.float32)
    r2 = r1 - h2
    return _dotd(sel, h1) + _dotd(sel, h2) + _dotd(sel, r2)


def _layers_body(node0, cart, ed, dstc, dstr,
                 ew1, eb1, ew2, eb2, nw1, nb1, nw2, nb2,
                 cw1, cb1, cw2, cb2,
                 node_out, coords_out,
                 node_s, coords_s, aggr_s, cdelta_s, ea_s):
    l = pl.program_id(0)
    e = pl.program_id(1)

    @pl.when((l == 0) & (e == 0))
    def _():
        node_s[...] = node0[...]
        coords_s[...] = cart[...]

    @pl.when(e == 0)
    def _():
        aggr_s[...] = jnp.zeros_like(aggr_s)
        cdelta_s[...] = jnp.zeros_like(cdelta_s)

    @pl.when(e < EB)
    def _():
        node = node_s[...]
        ns32 = node_s[pl.ds(e * SB, SB), :]
        cs32 = coords_s[pl.ds(e * SB, SB), :]
        coords = coords_s[...]

        srow = jax.lax.broadcasted_iota(jnp.int32, (EBS, SB), 0) // MAX_NEIGHBORS
        scol = jax.lax.broadcasted_iota(jnp.int32, (EBS, SB), 1)
        S = jnp.where(srow == scol, 1.0, 0.0).astype(jnp.float32)
        db = dstc[...].astype(jnp.int32)
        P = jnp.where(
            db == jax.lax.broadcasted_iota(jnp.int32, (EBS, N_ATOMS), 1),
            1.0, 0.0)
        dr = dstr[0].astype(jnp.int32)
        PT = jnp.where(
            dr == jax.lax.broadcasted_iota(jnp.int32, (N_ATOMS, EBS), 0),
            1.0, 0.0)

        nsrc = _dote(S, ns32)
        ndst = _dote(P, node)

        @pl.when(l == 0)
        def _():
            dd = ed[...]
            step = np.float32(CUTOFF) / np.float32(EDGE_DIM - 1)
            centers = jax.lax.broadcasted_iota(
                jnp.int32, (EBS, EDGE_DIM), 1).astype(jnp.float32) * step
            w2 = 2.0 * (CUTOFF / EDGE_DIM) ** 2
            rbf = jnp.exp(-((dd - centers) ** 2) / w2)
            env = 0.5 * (jnp.cos(dd * np.pi / CUTOFF) + 1.0) * (
                dd < CUTOFF).astype(jnp.float32)
            ea_s[pl.ds(e * EBS, EBS), :] = rbf * env

        ea = ea_s[pl.ds(e * EBS, EBS), :]

        w1 = ew1[0]
        m = (_dotd(nsrc, w1[:NODE_DIM])
             + _dotd(ndst, w1[NODE_DIM:2 * NODE_DIM])
             + _dotd(ea, w1[2 * NODE_DIM:])
             + eb1[0])
        m = _silu(m)
        m = _silu(_dotd(m, ew2[0]) + eb2[0])

        aggr_s[...] += _dote(PT, m)

        cw = _silu(_dotd(m, cw1[0]) + cb1[0])
        cw = _dotd(cw, cw2[0]) + cb2[0]

        csrc = _dote(S, cs32)
        cdst = _dote(P, coords)
        cd = csrc - cdst
        nrm = jnp.sqrt(jnp.sum(cd * cd, axis=1, keepdims=True))
        cdn = cd / (nrm + 1e-08)
        cdelta_s[...] += _dote(PT, cw * cdn)

    @pl.when(e == EB)
    def _():
        node = node_s[...]
        aggr = aggr_s[...]
        h = (_dotd(node, nw1[0, :NODE_DIM])
             + _dotd(aggr, nw1[0, NODE_DIM:])
             + nb1[0])
        nu = _dotd(_silu(h), nw2[0]) + nb2[0]
        node_s[...] = node + nu
        coords_s[...] = coords_s[...] + cdelta_s[...]

        @pl.when(l == NUM_LAYERS - 1)
        def _():
            node_out[...] = node_s[...]
            coords_out[...] = coords_s[...]


def kernel(atom_types, frac_coords, lengths, angles, timesteps, emb_table,
           time_W, time_b, edge_w1, edge_b1, edge_w2, edge_b2, node_w1,
           node_b1, node_w2, node_b2, coord_w1, coord_b1, coord_w2, coord_b2):
    f32 = jnp.float32
    lattice = _build_lattice(lengths, angles)
    cart = frac_coords @ lattice
    shifts = jnp.asarray(
        [[i, j, k] for i in (-1, 0, 1) for j in (-1, 0, 1) for k in (-1, 0, 1)],
        dtype=f32)
    shiftL = shifts @ lattice
    shifted = (cart[None, :, :] + shiftL[:, None, :]).reshape(NCAND, 3)
    sx = shifted[:, 0].reshape(1, NCAND)
    sy = shifted[:, 1].reshape(1, NCAND)
    sz = shifted[:, 2].reshape(1, NCAND)

    oh = (atom_types[:, None] == jnp.arange(128)[None, :]).astype(f32)
    emb128 = jnp.zeros((128, NODE_DIM), f32).at[:100].set(emb_table)

    dm, node0 = pl.pallas_call(
        _k1_body,
        out_shape=(jax.ShapeDtypeStruct((N_ATOMS, NCAND), f32),
                   jax.ShapeDtypeStruct((N_ATOMS, NODE_DIM), f32)),
    )(sx, sy, sz, cart, oh, emb128, timesteps, time_W, time_b.reshape(1, -1))

    dsel, isel = pl.pallas_call(
        _topk_body,
        out_shape=(jax.ShapeDtypeStruct((N_ATOMS, 128), f32),
                   jax.ShapeDtypeStruct((N_ATOMS, 128), jnp.int32)),
    )(dm)

    edist = jnp.broadcast_to(dsel[:, :MAX_NEIGHBORS].reshape(E, 1),
                             (E, EDGE_DIM))
    dst = (isel[:, :MAX_NEIGHBORS].reshape(-1) % N_ATOMS).astype(f32)
    dstc = dst.reshape(E, 1)
    dstr = dst.reshape(EB, 1, EBS)

    grid = (NUM_LAYERS, EB + 1)
    eb_map = lambda l, e: (jnp.minimum(e, EB - 1), 0)
    wmap2 = lambda l, e: (l, 0)
    wmap3 = lambda l, e: (l, 0, 0)
    cmap2 = lambda l, e: (0, 0)

    node_out, coords_out = pl.pallas_call(
        _layers_body,
        grid=grid,
        in_specs=[
            pl.BlockSpec((N_ATOMS, NODE_DIM), cmap2),
            pl.BlockSpec((N_ATOMS, 3), cmap2),
            pl.BlockSpec((EBS, EDGE_DIM), eb_map),
            pl.BlockSpec((EBS, 1), eb_map),
            pl.BlockSpec((1, 1, EBS), lambda l, e: (jnp.minimum(e, EB - 1), 0, 0)),
            pl.BlockSpec((1, 2 * NODE_DIM + EDGE_DIM, HIDDEN), wmap3),
            pl.BlockSpec((1, 1, HIDDEN), wmap3),
            pl.BlockSpec((1, HIDDEN, HIDDEN), wmap3),
            pl.BlockSpec((1, 1, HIDDEN), wmap3),
            pl.BlockSpec((1, NODE_DIM + HIDDEN, HIDDEN), wmap3),
            pl.BlockSpec((1, 1, HIDDEN), wmap3),
            pl.BlockSpec((1, HIDDEN, NODE_DIM), wmap3),
            pl.BlockSpec((1, 1, NODE_DIM), wmap3),
            pl.BlockSpec((1, HIDDEN, HIDDEN // 2), wmap3),
            pl.BlockSpec((1, 1, HIDDEN // 2), wmap3),
            pl.BlockSpec((1, HIDDEN // 2, 1), wmap3),
            pl.BlockSpec((1, 1, 1), wmap3),
        ],
        out_specs=[
            pl.BlockSpec((N_ATOMS, NODE_DIM), cmap2),
            pl.BlockSpec((N_ATOMS, 3), cmap2),
        ],
        out_shape=(jax.ShapeDtypeStruct((N_ATOMS, NODE_DIM), f32),
                   jax.ShapeDtypeStruct((N_ATOMS, 3), f32)),
        scratch_shapes=[
            pltpu.VMEM((N_ATOMS, NODE_DIM), f32),
            pltpu.VMEM((N_ATOMS, 3), f32),
            pltpu.VMEM((N_ATOMS, NODE_DIM), f32),
            pltpu.VMEM((N_ATOMS, 3), f32),
            pltpu.VMEM((E, EDGE_DIM), f32),
        ],
        compiler_params=pltpu.CompilerParams(
            dimension_semantics=("arbitrary", "arbitrary")),
    )(node0, cart, edist, dstc, dstr,
      edge_w1, edge_b1.reshape(NUM_LAYERS, 1, HIDDEN),
      edge_w2, edge_b2.reshape(NUM_LAYERS, 1, HIDDEN),
      node_w1, node_b1.reshape(NUM_LAYERS, 1, HIDDEN),
      node_w2, node_b2.reshape(NUM_LAYERS, 1, NODE_DIM),
      coord_w1, coord_b1.reshape(NUM_LAYERS, 1, HIDDEN // 2),
      coord_w2, coord_b2.reshape(NUM_LAYERS, 1, 1))

    return node_out, coords_out
